# baseline (speedup 1.0000x reference)
import functools

import jax
import jax.numpy as jnp
from jax import lax
from jax.experimental import pallas as pl
from jax.experimental.pallas import tpu as pltpu
from jax.experimental.pallas import tpu_sc as plsc

N_NODES = 10000
N_EDGES = 320000
D = 128

NC = 2
NS = 16
CHUNK = 128
SLAB = 32
NSLAB0 = 5
NSLAB1 = 0
NCH0 = NSLAB0 * SLAB
NCH1 = NSLAB1 * SLAB
E_PAD = NS * (NCH0 + NCH1) * CHUNK
NCHT = E_PAD // CHUNK
N_PAD = 10240
ROWS_PER_TILE = N_PAD // NS
NBUF = 2

_mesh = plsc.VectorSubcoreMesh(
    core_axis_name="c", subcore_axis_name="s", num_cores=NC, num_subcores=NS
)


@functools.partial(
    pl.kernel,
    out_type=jax.ShapeDtypeStruct((NC, N_PAD, D), jnp.float32),
    mesh=_mesh,
    scratch_types=[
        pltpu.VMEM_SHARED((N_PAD, D), jnp.float32),
        pltpu.VMEM((SLAB * CHUNK,), jnp.int32),
        pltpu.VMEM((SLAB, CHUNK), jnp.int32),
        [pltpu.VMEM((CHUNK, D), jnp.float32)] * NBUF,
        pltpu.SemaphoreType.DMA,
    ],
)
def _sc_segment_sum(x_hbm, src_hbm, dst_hbm, out_hbm, acc, srcs_v, dsts_v,
                    rows, sem_g):
    c = lax.axis_index("c")
    s = lax.axis_index("s")

    @pl.loop(0, CHUNK)
    def _(i):
        @pl.loop(0, D, step=16)
        def _(j):
            rows[0][i, pl.ds(j, 16)] = jnp.zeros((16,), jnp.float32)

    row0 = s * ROWS_PER_TILE

    @pl.loop(0, ROWS_PER_TILE, step=CHUNK)
    def _(r):
        pltpu.sync_copy(rows[0], acc.at[pl.ds(row0 + r, CHUNK)])

    plsc.subcore_barrier()

    def gather(j, b):
        pltpu.async_copy(
            x_hbm.at[srcs_v.at[pl.ds(j * CHUNK, CHUNK)]], rows[b], sem_g)

    def wait_gather(j, b):
        pltpu.make_async_copy(
            x_hbm.at[srcs_v.at[pl.ds(j * CHUNK, CHUNK)]], rows[b],
            sem_g).wait()

    def run_slabs(chunk0, nslab, nbuf):
        for k in range(nslab):
            srow = chunk0 + k * SLAB
            pltpu.sync_copy(src_hbm.at[pl.ds(srow * CHUNK, SLAB * CHUNK)],
                            srcs_v)
            pltpu.sync_copy(dst_hbm.at[pl.ds(srow, SLAB)], dsts_v)
            if nbuf == 1:
                @pl.loop(0, SLAB)
                def _(j):
                    gather(j, 0)
                    wait_gather(j, 0)
                    pltpu.sync_copy(rows[0], acc.at[dsts_v.at[j]], add=True)
            else:
                for b in range(nbuf):
                    gather(b, b)

                @pl.loop(0, (SLAB - nbuf) // nbuf)
                def _(go):
                    for b in range(nbuf):
                        j = go * nbuf + b
                        wait_gather(j, b)
                        pltpu.sync_copy(rows[b], acc.at[dsts_v.at[j]],
                                        add=True)
                        gather(j + nbuf, b)

                for b in range(nbuf):
                    j = SLAB - nbuf + b
                    wait_gather(j, b)
                    pltpu.sync_copy(rows[b], acc.at[dsts_v.at[j]], add=True)

    @pl.when(c == 0)
    def _():
        run_slabs(s * NCH0, NSLAB0, NBUF)

    @pl.when(c == 1)
    def _():
        run_slabs(NS * NCH0 + s * NCH1, NSLAB1, 1)

    plsc.subcore_barrier()

    pltpu.sync_copy(acc.at[pl.ds(row0, ROWS_PER_TILE)],
                    out_hbm.at[c, pl.ds(row0, ROWS_PER_TILE)])


_BR = 2048


def _mm_body(p_ref, wt_ref, o_ref):
    h = p_ref[0] + p_ref[1]
    o_ref[...] = jax.lax.dot(h, wt_ref[...],
                             precision=jax.lax.Precision.HIGHEST,
                             preferred_element_type=jnp.float32)


def _tc_linear(partial, wt):
    return pl.pallas_call(
        _mm_body,
        out_shape=jax.ShapeDtypeStruct((N_PAD, D), jnp.float32),
        grid=(N_PAD // _BR,),
        in_specs=[
            pl.BlockSpec((NC, _BR, D), lambda r: (0, r, 0)),
            pl.BlockSpec((D, D), lambda r: (0, 0)),
        ],
        out_specs=pl.BlockSpec((_BR, D), lambda r: (r, 0)),
    )(partial, wt)


def kernel(x, edge_index, W):
    ei = edge_index.astype(jnp.int32)
    pad = jnp.stack([
        jnp.zeros((E_PAD - N_EDGES,), jnp.int32),
        jnp.full((E_PAD - N_EDGES,), N_NODES, jnp.int32),
    ])
    ei = jnp.concatenate([ei, pad], axis=1)
    partial = _sc_segment_sum(x, ei[0], ei[1].reshape(NCHT, CHUNK))
    return _tc_linear(partial, W.T)[:N_NODES]


# speedup vs baseline: 3.9267x; 3.9267x over previous
import functools

import jax
import jax.numpy as jnp
from jax import lax
from jax.experimental import pallas as pl
from jax.experimental.pallas import tpu as pltpu
from jax.experimental.pallas import tpu_sc as plsc

N_NODES = 10000
N_EDGES = 320000
D = 128

NC = 2
NS = 16
CHUNK = 128
SLAB = 40
NSLAB0 = 2
NSLAB1 = 2
NCH0 = NSLAB0 * SLAB
NCH1 = NSLAB1 * SLAB
E_PAD = NS * (NCH0 + NCH1) * CHUNK
NCHT = E_PAD // CHUNK
N_PAD = 10240
ROWS_PER_TILE = N_PAD // NS
NBUF = 2

_mesh = plsc.VectorSubcoreMesh(
    core_axis_name="c", subcore_axis_name="s", num_cores=NC, num_subcores=NS
)


@functools.partial(
    pl.kernel,
    out_type=jax.ShapeDtypeStruct((NC, N_PAD, D), jnp.float32),
    mesh=_mesh,
    scratch_types=[
        pltpu.VMEM_SHARED((N_PAD, D), jnp.float32),
        pltpu.VMEM((SLAB * CHUNK,), jnp.int32),
        pltpu.VMEM((SLAB, CHUNK), jnp.int32),
        [pltpu.VMEM((CHUNK, D), jnp.float32)] * NBUF,
        pltpu.SemaphoreType.DMA,
    ],
)
def _sc_segment_sum(x_hbm, src_hbm, dst_hbm, out_hbm, acc, srcs_v, dsts_v,
                    rows, sem_g):
    c = lax.axis_index("c")
    s = lax.axis_index("s")

    @pl.loop(0, CHUNK)
    def _(i):
        @pl.loop(0, D, step=16)
        def _(j):
            rows[0][i, pl.ds(j, 16)] = jnp.zeros((16,), jnp.float32)

    row0 = s * ROWS_PER_TILE

    @pl.loop(0, ROWS_PER_TILE, step=CHUNK)
    def _(r):
        pltpu.sync_copy(rows[0], acc.at[pl.ds(row0 + r, CHUNK)])

    plsc.subcore_barrier()

    def gather(j, b):
        pltpu.async_copy(
            x_hbm.at[srcs_v.at[pl.ds(j * CHUNK, CHUNK)]], rows[b], sem_g)

    def wait_gather(j, b):
        pltpu.make_async_copy(
            x_hbm.at[srcs_v.at[pl.ds(j * CHUNK, CHUNK)]], rows[b],
            sem_g).wait()

    def run_slabs(chunk0, nslab, nbuf):
        for k in range(nslab):
            srow = chunk0 + k * SLAB
            pltpu.sync_copy(src_hbm.at[pl.ds(srow * CHUNK, SLAB * CHUNK)],
                            srcs_v)
            pltpu.sync_copy(dst_hbm.at[pl.ds(srow, SLAB)], dsts_v)
            if nbuf == 1:
                @pl.loop(0, SLAB)
                def _(j):
                    gather(j, 0)
                    wait_gather(j, 0)
                    pltpu.sync_copy(rows[0], acc.at[dsts_v.at[j]], add=True)
            else:
                for b in range(nbuf):
                    gather(b, b)

                @pl.loop(0, (SLAB - nbuf) // nbuf)
                def _(go):
                    for b in range(nbuf):
                        j = go * nbuf + b
                        wait_gather(j, b)
                        pltpu.sync_copy(rows[b], acc.at[dsts_v.at[j]],
                                        add=True)
                        gather(j + nbuf, b)

                for b in range(nbuf):
                    j = SLAB - nbuf + b
                    wait_gather(j, b)
                    pltpu.sync_copy(rows[b], acc.at[dsts_v.at[j]], add=True)

    @pl.when(c == 0)
    def _():
        run_slabs(s * NCH0, NSLAB0, NBUF)

    @pl.when(c == 1)
    def _():
        run_slabs(NS * NCH0 + s * NCH1, NSLAB1, NBUF)

    plsc.subcore_barrier()

    pltpu.sync_copy(acc.at[pl.ds(row0, ROWS_PER_TILE)],
                    out_hbm.at[c, pl.ds(row0, ROWS_PER_TILE)])


_BR = 2048


def _mm_body(p_ref, wt_ref, o_ref):
    h = p_ref[0] + p_ref[1]
    o_ref[...] = jax.lax.dot(h, wt_ref[...],
                             precision=jax.lax.Precision.HIGHEST,
                             preferred_element_type=jnp.float32)


def _tc_linear(partial, wt):
    return pl.pallas_call(
        _mm_body,
        out_shape=jax.ShapeDtypeStruct((N_PAD, D), jnp.float32),
        grid=(N_PAD // _BR,),
        in_specs=[
            pl.BlockSpec((NC, _BR, D), lambda r: (0, r, 0)),
            pl.BlockSpec((D, D), lambda r: (0, 0)),
        ],
        out_specs=pl.BlockSpec((_BR, D), lambda r: (r, 0)),
    )(partial, wt)


def kernel(x, edge_index, W):
    ei = edge_index.astype(jnp.int32)
    npad = E_PAD - N_EDGES
    it = jnp.arange(npad, dtype=jnp.int32)
    pad = jnp.stack([
        it % N_NODES,
        N_NODES + it % (N_PAD - N_NODES),
    ])
    ei = jnp.concatenate([ei, pad], axis=1)
    partial = _sc_segment_sum(x, ei[0], ei[1].reshape(NCHT, CHUNK))
    return _tc_linear(partial, W.T)[:N_NODES]
